# Initial kernel scaffold; baseline (speedup 1.0000x reference)
import jax
import jax.numpy as jnp
from jax.experimental import pallas as pl


def kernel(positions, asset_ids, sector_ids, market_caps):
    raise NotImplementedError("write your pallas kernel here")


# speedup vs baseline: 9.9557x; 9.9557x over previous
import functools

import jax
import jax.numpy as jnp
from jax import lax
from jax.experimental import pallas as pl
from jax.experimental.pallas import tpu as pltpu
from jax.experimental.pallas import tpu_sc as plsc

N_TOTAL = 5_000_000
NUM_SECTORS = 11
MAX_INDIVIDUAL = 0.01
MAX_SECTOR = 0.15
VOL_SCALAR = 0.15 / 0.2

BLK = 8_000
NBLK = N_TOTAL // BLK
NC, NS, L = 2, 16, 16
NW = NC * NS
ACC_ROWS = 24
ACC_LEN = ACC_ROWS * L
PART_LEN = NW * ACC_LEN

_MASK_HI = -65536
_MAGIC = 0x5F3759DF


def _rsqrt_newton(x):
    bits = lax.bitcast_convert_type(x, jnp.int32)
    y = lax.bitcast_convert_type(_MAGIC - lax.shift_right_logical(bits, 1), jnp.float32)
    th = x * 0.5
    y = y * (1.5 - th * y * y)
    y = y * (1.5 - th * y * y)
    return y


def _num_blocks(wid):
    rem = NBLK % NW
    return jnp.where(wid < rem, NBLK // NW + 1, NBLK // NW)


def _pass1_body(pos_hbm, sec_hbm, mc_hbm, packed_hbm, part_hbm,
                pos_v, sec_v, mc_v, pck_v, acc):
    wid = lax.axis_index("s") * NC + lax.axis_index("c")
    zeros = jnp.zeros((L,), jnp.float32)
    for r in range(ACC_ROWS):
        acc[pl.ds(r * L, L)] = zeros
    iota = lax.iota(jnp.int32, L)

    def chunk(j, carry):
        o = j * L
        pos = pos_v[pl.ds(o, L)]
        sec = sec_v[pl.ds(o, L)]
        mc = mc_v[pl.ds(o, L)]
        clamped = jnp.clip(pos * VOL_SCALAR, -MAX_INDIVIDUAL, MAX_INDIVIDUAL)
        absc = jnp.abs(clamped)
        x = jnp.maximum(mc, 1e-12)
        s = x * _rsqrt_newton(x)
        u = clamped * s
        absu = jnp.abs(u)
        idx_a = (sec << 4) + iota
        plsc.addupdate_scatter(acc, [idx_a], absc)
        plsc.addupdate_scatter(acc, [idx_a + (NUM_SECTORS * L)], absu)
        plsc.addupdate(acc.at[pl.ds(2 * NUM_SECTORS * L, L)], mc)
        ub = lax.bitcast_convert_type(u, jnp.int32)
        pck = ((ub + 32768) & _MASK_HI) | sec
        pck_v[pl.ds(o, L)] = pck
        return carry

    def blk_body(t, carry):
        base = (wid + t * NW) * BLK
        pltpu.sync_copy(pos_hbm.at[pl.ds(base, BLK)], pos_v)
        pltpu.sync_copy(sec_hbm.at[pl.ds(base, BLK)], sec_v)
        pltpu.sync_copy(mc_hbm.at[pl.ds(base, BLK)], mc_v)
        lax.fori_loop(0, BLK // L, chunk, 0, unroll=4)
        pltpu.sync_copy(pck_v, packed_hbm.at[pl.ds(base, BLK)])
        return carry

    lax.fori_loop(0, _num_blocks(wid), blk_body, 0)
    pltpu.sync_copy(acc, part_hbm.at[pl.ds(wid * ACC_LEN, ACC_LEN)])


def _pass2_body(packed_hbm, part_hbm, out_hbm, pck_v, out_v, part_v, lut):
    wid = lax.axis_index("s") * NC + lax.axis_index("c")
    pltpu.sync_copy(part_hbm, part_v)
    iota = lax.iota(jnp.int32, L)
    fiota = iota.astype(jnp.float32)

    rows = []
    for r in range(2 * NUM_SECTORS + 1):
        v = part_v[pl.ds(r * L, L)]
        for w in range(1, NW):
            v = v + part_v[pl.ds(w * ACC_LEN + r * L, L)]
        rows.append(v)

    a_scalars = [jnp.sum(rows[r]) for r in range(NUM_SECTORS)]
    b_scalars = [jnp.sum(rows[NUM_SECTORS + r]) for r in range(NUM_SECTORS)]
    c_total = jnp.sum(rows[2 * NUM_SECTORS])

    zero = jnp.zeros((L,), jnp.float32)
    a_vec = zero
    b_vec = zero
    for r in range(NUM_SECTORS):
        sel = fiota == float(r)
        a_vec = jnp.where(sel, a_scalars[r], a_vec)
        b_vec = jnp.where(sel, b_scalars[r], b_vec)

    scale_v = jnp.where(a_vec > MAX_SECTOR, MAX_SECTOR / a_vec, 1.0)
    c_vec = jnp.maximum(jnp.zeros((L,), jnp.float32) + c_total, 1e-12)
    rc_v = _rsqrt_newton(c_vec)
    total_v = jnp.zeros((L,), jnp.float32) + jnp.sum(scale_v * b_vec * rc_v)
    k_vec = jnp.where(total_v > 1.0, 1.0 / total_v, 1.0)
    lut[...] = scale_v * rc_v * k_vec

    def chunk(j, carry):
        o = j * L
        x = pck_v[pl.ds(o, L)]
        u = lax.bitcast_convert_type(x & _MASK_HI, jnp.float32)
        sidx = x & 15
        g = plsc.load_gather(lut, [sidx])
        out_v[pl.ds(o, L)] = u * g
        return carry

    def blk_body(t, carry):
        base = (wid + t * NW) * BLK
        pltpu.sync_copy(packed_hbm.at[pl.ds(base, BLK)], pck_v)
        lax.fori_loop(0, BLK // L, chunk, 0, unroll=8)
        pltpu.sync_copy(out_v, out_hbm.at[pl.ds(base, BLK)])
        return carry

    lax.fori_loop(0, _num_blocks(wid), blk_body, 0)


_MESH = plsc.VectorSubcoreMesh(
    core_axis_name="c", subcore_axis_name="s", num_cores=NC, num_subcores=NS)
_PARAMS = pltpu.CompilerParams(needs_layout_passes=False)

_pass1 = pl.kernel(
    _pass1_body,
    out_type=(
        jax.ShapeDtypeStruct((N_TOTAL,), jnp.int32),
        jax.ShapeDtypeStruct((PART_LEN,), jnp.float32),
    ),
    mesh=_MESH,
    compiler_params=_PARAMS,
    scratch_types=[
        pltpu.VMEM((BLK,), jnp.float32),
        pltpu.VMEM((BLK,), jnp.int32),
        pltpu.VMEM((BLK,), jnp.float32),
        pltpu.VMEM((BLK,), jnp.int32),
        pltpu.VMEM((ACC_LEN,), jnp.float32),
    ],
)

_pass2 = pl.kernel(
    _pass2_body,
    out_type=jax.ShapeDtypeStruct((N_TOTAL,), jnp.float32),
    mesh=_MESH,
    compiler_params=_PARAMS,
    scratch_types=[
        pltpu.VMEM((BLK,), jnp.int32),
        pltpu.VMEM((BLK,), jnp.float32),
        pltpu.VMEM((PART_LEN,), jnp.float32),
        pltpu.VMEM((L,), jnp.float32),
    ],
)


def kernel(positions, asset_ids, sector_ids, market_caps):
    del asset_ids
    pos = positions.reshape(N_TOTAL)
    packed, part = _pass1(pos, sector_ids, market_caps)
    out = _pass2(packed, part)
    return out.reshape(N_TOTAL, 1)


# speedup vs baseline: 16.9588x; 1.7034x over previous
import functools

import jax
import jax.numpy as jnp
from jax import lax
from jax.experimental import pallas as pl
from jax.experimental.pallas import tpu as pltpu
from jax.experimental.pallas import tpu_sc as plsc

N_TOTAL = 5_000_000
NUM_SECTORS = 11
MAX_INDIVIDUAL = 0.01
MAX_SECTOR = 0.15
VOL_SCALAR = 0.15 / 0.2

BLK = 8_192
NBLK = N_TOTAL // BLK
TAIL_BASE = NBLK * BLK
TAIL = N_TOTAL - TAIL_BASE
NC, NS, L = 2, 16, 16
NW = NC * NS
ACC_ROWS = 24
ACC_LEN = ACC_ROWS * L
PART_LEN = NW * ACC_LEN

_MASK_HI = -65536
_MAGIC = 0x5F3759DF


def _rsqrt_newton(x):
    bits = lax.bitcast_convert_type(x, jnp.int32)
    y = lax.bitcast_convert_type(_MAGIC - lax.shift_right_logical(bits, 1), jnp.float32)
    th = x * 0.5
    y = y * (1.5 - th * y * y)
    y = y * (1.5 - th * y * y)
    return y


def _num_blocks(wid):
    rem = NBLK % NW
    return jnp.where(wid < rem, NBLK // NW + 1, NBLK // NW)


def _pass1_body(pos_hbm, sec_hbm, mc_hbm, packed_hbm, part_hbm,
                pos_v, sec_v, mc_v, pck_v, acc):
    wid = lax.axis_index("s") * NC + lax.axis_index("c")
    zeros = jnp.zeros((L,), jnp.float32)
    for r in range(ACC_ROWS):
        acc[pl.ds(r * L, L)] = zeros
    iota = lax.iota(jnp.int32, L)

    def chunk(j, carry):
        o = j * L
        pos = pos_v[pl.ds(o, L)]
        sec = sec_v[pl.ds(o, L)]
        mc = mc_v[pl.ds(o, L)]
        clamped = jnp.clip(pos * VOL_SCALAR, -MAX_INDIVIDUAL, MAX_INDIVIDUAL)
        absc = jnp.abs(clamped)
        x = jnp.maximum(mc, 1e-12)
        s = x * _rsqrt_newton(x)
        u = clamped * s
        absu = jnp.abs(u)
        idx_a = (sec << 4) + iota
        plsc.addupdate_scatter(acc, [idx_a], absc)
        plsc.addupdate_scatter(acc, [idx_a + (NUM_SECTORS * L)], absu)
        plsc.addupdate(acc.at[pl.ds(2 * NUM_SECTORS * L, L)], mc)
        ub = lax.bitcast_convert_type(u, jnp.int32)
        pck = ((ub + 32768) & _MASK_HI) | sec
        pck_v[pl.ds(o, L)] = pck
        return carry

    def do_block(base, nelem):
        pltpu.sync_copy(pos_hbm.at[0, pl.ds(base, nelem)], pos_v.at[pl.ds(0, nelem)])
        pltpu.sync_copy(sec_hbm.at[pl.ds(base, nelem)], sec_v.at[pl.ds(0, nelem)])
        pltpu.sync_copy(mc_hbm.at[pl.ds(base, nelem)], mc_v.at[pl.ds(0, nelem)])
        lax.fori_loop(0, nelem // L, chunk, 0, unroll=4)
        pltpu.sync_copy(pck_v.at[pl.ds(0, nelem)], packed_hbm.at[pl.ds(base, nelem)])

    def blk_body(t, carry):
        do_block(pl.multiple_of((wid + t * NW) * BLK, 128), BLK)
        return carry

    lax.fori_loop(0, _num_blocks(wid), blk_body, 0)

    @pl.when(wid == NW - 1)
    def _():
        do_block(TAIL_BASE, TAIL)

    pltpu.sync_copy(acc, part_hbm.at[pl.ds(wid * ACC_LEN, ACC_LEN)])


def _pass2_body(packed_hbm, part_hbm, out_hbm, pck_v, out_v, part_v, lut):
    wid = lax.axis_index("s") * NC + lax.axis_index("c")
    pltpu.sync_copy(part_hbm, part_v)
    iota = lax.iota(jnp.int32, L)
    fiota = iota.astype(jnp.float32)

    rows = []
    for r in range(2 * NUM_SECTORS + 1):
        v = part_v[pl.ds(r * L, L)]
        for w in range(1, NW):
            v = v + part_v[pl.ds(w * ACC_LEN + r * L, L)]
        rows.append(v)

    a_scalars = [jnp.sum(rows[r]) for r in range(NUM_SECTORS)]
    b_scalars = [jnp.sum(rows[NUM_SECTORS + r]) for r in range(NUM_SECTORS)]
    c_total = jnp.sum(rows[2 * NUM_SECTORS])

    zero = jnp.zeros((L,), jnp.float32)
    a_vec = zero
    b_vec = zero
    for r in range(NUM_SECTORS):
        sel = fiota == float(r)
        a_vec = jnp.where(sel, a_scalars[r], a_vec)
        b_vec = jnp.where(sel, b_scalars[r], b_vec)

    scale_v = jnp.where(a_vec > MAX_SECTOR, MAX_SECTOR / a_vec, 1.0)
    c_vec = jnp.maximum(jnp.zeros((L,), jnp.float32) + c_total, 1e-12)
    rc_v = _rsqrt_newton(c_vec)
    total_v = jnp.zeros((L,), jnp.float32) + jnp.sum(scale_v * b_vec * rc_v)
    k_vec = jnp.where(total_v > 1.0, 1.0 / total_v, 1.0)
    lut[...] = scale_v * rc_v * k_vec

    def chunk(j, carry):
        o = j * L
        x = pck_v[pl.ds(o, L)]
        u = lax.bitcast_convert_type(x & _MASK_HI, jnp.float32)
        sidx = x & 15
        g = plsc.load_gather(lut, [sidx])
        out_v[pl.ds(o, L)] = u * g
        return carry

    def do_block(base, nelem):
        pltpu.sync_copy(packed_hbm.at[pl.ds(base, nelem)], pck_v.at[pl.ds(0, nelem)])
        lax.fori_loop(0, nelem // L, chunk, 0, unroll=8)
        pltpu.sync_copy(out_v.at[pl.ds(0, nelem)], out_hbm.at[0, pl.ds(base, nelem)])

    def blk_body(t, carry):
        do_block(pl.multiple_of((wid + t * NW) * BLK, 128), BLK)
        return carry

    lax.fori_loop(0, _num_blocks(wid), blk_body, 0)

    @pl.when(wid == NW - 1)
    def _():
        do_block(TAIL_BASE, TAIL)


_MESH = plsc.VectorSubcoreMesh(
    core_axis_name="c", subcore_axis_name="s", num_cores=NC, num_subcores=NS)
_PARAMS = pltpu.CompilerParams(needs_layout_passes=False)

_pass1 = pl.kernel(
    _pass1_body,
    out_type=(
        jax.ShapeDtypeStruct((N_TOTAL,), jnp.int32),
        jax.ShapeDtypeStruct((PART_LEN,), jnp.float32),
    ),
    mesh=_MESH,
    compiler_params=_PARAMS,
    scratch_types=[
        pltpu.VMEM((BLK,), jnp.float32),
        pltpu.VMEM((BLK,), jnp.int32),
        pltpu.VMEM((BLK,), jnp.float32),
        pltpu.VMEM((BLK,), jnp.int32),
        pltpu.VMEM((ACC_LEN,), jnp.float32),
    ],
)

_pass2 = pl.kernel(
    _pass2_body,
    out_type=jax.ShapeDtypeStruct((1, N_TOTAL), jnp.float32),
    mesh=_MESH,
    compiler_params=_PARAMS,
    scratch_types=[
        pltpu.VMEM((BLK,), jnp.int32),
        pltpu.VMEM((BLK,), jnp.float32),
        pltpu.VMEM((PART_LEN,), jnp.float32),
        pltpu.VMEM((L,), jnp.float32),
    ],
)


def kernel(positions, asset_ids, sector_ids, market_caps):
    del asset_ids
    pos_1n = positions.reshape(1, N_TOTAL)
    packed, part = _pass1(pos_1n, sector_ids, market_caps)
    return _pass2(packed, part).reshape(N_TOTAL, 1)


# speedup vs baseline: 21.3921x; 1.2614x over previous
import functools

import jax
import jax.numpy as jnp
from jax import lax
from jax.experimental import pallas as pl
from jax.experimental.pallas import tpu as pltpu
from jax.experimental.pallas import tpu_sc as plsc

N_TOTAL = 5_000_000
NUM_SECTORS = 11
MAX_INDIVIDUAL = 0.01
MAX_SECTOR = 0.15
VOL_SCALAR = 0.15 / 0.2

BLK = 8_192
NBLK = N_TOTAL // BLK
TAIL_BASE = NBLK * BLK
TAIL = N_TOTAL - TAIL_BASE
NC, NS, L = 2, 16, 16
NW = NC * NS
ACC_ROWS = 24
ACC_LEN = ACC_ROWS * L
PART_LEN = NW * ACC_LEN

_MASK_HI = -65536
_MAGIC = 0x5F3759DF


def _rsqrt_newton(x):
    bits = lax.bitcast_convert_type(x, jnp.int32)
    y = lax.bitcast_convert_type(_MAGIC - lax.shift_right_logical(bits, 1), jnp.float32)
    th = x * 0.5
    y = y * (1.5 - th * y * y)
    y = y * (1.5 - th * y * y)
    return y


def _num_blocks(wid):
    rem = NBLK % NW
    return jnp.where(wid < rem, NBLK // NW + 1, NBLK // NW)


def _pass1_body(pos_hbm, sec_hbm, mc_hbm, packed_hbm, part_hbm,
                pos0, pos1, sec0, sec1, mc0, mc1, pck0, pck1, acc,
                si0, si1, so0, so1):
    wid = lax.axis_index("s") * NC + lax.axis_index("c")
    zeros = jnp.zeros((L,), jnp.float32)
    for r in range(ACC_ROWS):
        acc[pl.ds(r * L, L)] = zeros
    iota = lax.iota(jnp.int32, L)
    slots = ((pos0, sec0, mc0, pck0, si0, so0),
             (pos1, sec1, mc1, pck1, si1, so1))
    nb = _num_blocks(wid)

    def base_of(t):
        return pl.multiple_of((wid + t * NW) * BLK, 128)

    def start_in(t, s):
        b = base_of(t)
        pltpu.async_copy(pos_hbm.at[0, pl.ds(b, BLK)], s[0], s[4])
        pltpu.async_copy(sec_hbm.at[pl.ds(b, BLK)], s[1], s[4])
        pltpu.async_copy(mc_hbm.at[pl.ds(b, BLK)], s[2], s[4])

    def wait_in(s):
        pltpu.make_async_copy(pos_hbm.at[0, pl.ds(0, BLK)], s[0], s[4]).wait()
        pltpu.make_async_copy(sec_hbm.at[pl.ds(0, BLK)], s[1], s[4]).wait()
        pltpu.make_async_copy(mc_hbm.at[pl.ds(0, BLK)], s[2], s[4]).wait()

    def wait_out(s):
        pltpu.make_async_copy(s[3], packed_hbm.at[pl.ds(0, BLK)], s[5]).wait()

    def process(s, nelem):
        pos_v, sec_v, mc_v, pck_v = s[0], s[1], s[2], s[3]

        def chunk(j, carry):
            o = j * L
            pos = pos_v[pl.ds(o, L)]
            sec = sec_v[pl.ds(o, L)]
            mc = mc_v[pl.ds(o, L)]
            clamped = jnp.clip(pos * VOL_SCALAR, -MAX_INDIVIDUAL, MAX_INDIVIDUAL)
            absc = jnp.abs(clamped)
            x = jnp.maximum(mc, 1e-12)
            sq = x * _rsqrt_newton(x)
            u = clamped * sq
            absu = absc * sq
            idx_a = (sec << 4) + iota
            plsc.addupdate_scatter(acc, [idx_a], absc)
            plsc.addupdate_scatter(acc, [idx_a + (NUM_SECTORS * L)], absu)
            plsc.addupdate(acc.at[pl.ds(2 * NUM_SECTORS * L, L)], mc)
            ub = lax.bitcast_convert_type(u, jnp.int32)
            pck = ((ub + 32768) & _MASK_HI) | sec
            pck_v[pl.ds(o, L)] = pck
            return carry

        lax.fori_loop(0, nelem // L, chunk, 0, unroll=8)

    start_in(0, slots[0])

    def blk_body(t, carry):
        for sl in (0, 1):
            @pl.when((t & 1) == sl)
            def _():
                s = slots[sl]
                wait_in(s)

                @pl.when(t + 1 < nb)
                def _():
                    start_in(t + 1, slots[1 - sl])

                @pl.when(t >= 2)
                def _():
                    wait_out(s)

                process(s, BLK)
                pltpu.async_copy(s[3], packed_hbm.at[pl.ds(base_of(t), BLK)], s[5])
        return carry

    lax.fori_loop(0, nb, blk_body, 0)
    wait_out(slots[0])
    wait_out(slots[1])

    @pl.when(wid == NW - 1)
    def _():
        s = slots[0]
        pltpu.sync_copy(pos_hbm.at[0, pl.ds(TAIL_BASE, TAIL)], s[0].at[pl.ds(0, TAIL)])
        pltpu.sync_copy(sec_hbm.at[pl.ds(TAIL_BASE, TAIL)], s[1].at[pl.ds(0, TAIL)])
        pltpu.sync_copy(mc_hbm.at[pl.ds(TAIL_BASE, TAIL)], s[2].at[pl.ds(0, TAIL)])
        process(s, TAIL)
        pltpu.sync_copy(s[3].at[pl.ds(0, TAIL)], packed_hbm.at[pl.ds(TAIL_BASE, TAIL)])

    pltpu.sync_copy(acc, part_hbm.at[pl.ds(wid * ACC_LEN, ACC_LEN)])


def _pass2_body(packed_hbm, part_hbm, out_hbm,
                pck0, pck1, out0, out1, part_v, lut, si0, si1, so0, so1):
    wid = lax.axis_index("s") * NC + lax.axis_index("c")
    pltpu.sync_copy(part_hbm, part_v)
    iota = lax.iota(jnp.int32, L)
    fiota = iota.astype(jnp.float32)

    rows = []
    for r in range(2 * NUM_SECTORS + 1):
        v = part_v[pl.ds(r * L, L)]
        for w in range(1, NW):
            v = v + part_v[pl.ds(w * ACC_LEN + r * L, L)]
        rows.append(v)

    a_scalars = [jnp.sum(rows[r]) for r in range(NUM_SECTORS)]
    b_scalars = [jnp.sum(rows[NUM_SECTORS + r]) for r in range(NUM_SECTORS)]
    c_total = jnp.sum(rows[2 * NUM_SECTORS])

    zero = jnp.zeros((L,), jnp.float32)
    a_vec = zero
    b_vec = zero
    for r in range(NUM_SECTORS):
        sel = fiota == float(r)
        a_vec = jnp.where(sel, a_scalars[r], a_vec)
        b_vec = jnp.where(sel, b_scalars[r], b_vec)

    scale_v = jnp.where(a_vec > MAX_SECTOR, MAX_SECTOR / a_vec, 1.0)
    c_vec = jnp.maximum(jnp.zeros((L,), jnp.float32) + c_total, 1e-12)
    rc_v = _rsqrt_newton(c_vec)
    total_v = jnp.zeros((L,), jnp.float32) + jnp.sum(scale_v * b_vec * rc_v)
    k_vec = jnp.where(total_v > 1.0, 1.0 / total_v, 1.0)
    lut[...] = scale_v * rc_v * k_vec

    slots = ((pck0, out0, si0, so0), (pck1, out1, si1, so1))
    nb = _num_blocks(wid)

    def base_of(t):
        return pl.multiple_of((wid + t * NW) * BLK, 128)

    def process(s, nelem):
        pck_v, out_v = s[0], s[1]

        def chunk(j, carry):
            o = j * L
            x = pck_v[pl.ds(o, L)]
            u = lax.bitcast_convert_type(x & _MASK_HI, jnp.float32)
            sidx = x & 15
            g = plsc.load_gather(lut, [sidx])
            out_v[pl.ds(o, L)] = u * g
            return carry

        lax.fori_loop(0, nelem // L, chunk, 0, unroll=8)

    def wait_in(s):
        pltpu.make_async_copy(packed_hbm.at[pl.ds(0, BLK)], s[0], s[2]).wait()

    def wait_out(s):
        pltpu.make_async_copy(s[1], out_hbm.at[0, pl.ds(0, BLK)], s[3]).wait()

    pltpu.async_copy(packed_hbm.at[pl.ds(base_of(0), BLK)], slots[0][0], slots[0][2])

    def blk_body(t, carry):
        for sl in (0, 1):
            @pl.when((t & 1) == sl)
            def _():
                s = slots[sl]
                wait_in(s)

                @pl.when(t + 1 < nb)
                def _():
                    sn = slots[1 - sl]
                    pltpu.async_copy(
                        packed_hbm.at[pl.ds(base_of(t + 1), BLK)], sn[0], sn[2])

                @pl.when(t >= 2)
                def _():
                    wait_out(s)

                process(s, BLK)
                pltpu.async_copy(s[1], out_hbm.at[0, pl.ds(base_of(t), BLK)], s[3])
        return carry

    lax.fori_loop(0, nb, blk_body, 0)
    wait_out(slots[0])
    wait_out(slots[1])

    @pl.when(wid == NW - 1)
    def _():
        s = slots[0]
        pltpu.sync_copy(packed_hbm.at[pl.ds(TAIL_BASE, TAIL)], s[0].at[pl.ds(0, TAIL)])
        process(s, TAIL)
        pltpu.sync_copy(s[1].at[pl.ds(0, TAIL)], out_hbm.at[0, pl.ds(TAIL_BASE, TAIL)])


_MESH = plsc.VectorSubcoreMesh(
    core_axis_name="c", subcore_axis_name="s", num_cores=NC, num_subcores=NS)
_PARAMS = pltpu.CompilerParams(needs_layout_passes=False)

_pass1 = pl.kernel(
    _pass1_body,
    out_type=(
        jax.ShapeDtypeStruct((N_TOTAL,), jnp.int32),
        jax.ShapeDtypeStruct((PART_LEN,), jnp.float32),
    ),
    mesh=_MESH,
    compiler_params=_PARAMS,
    scratch_types=[
        pltpu.VMEM((BLK,), jnp.float32), pltpu.VMEM((BLK,), jnp.float32),
        pltpu.VMEM((BLK,), jnp.int32), pltpu.VMEM((BLK,), jnp.int32),
        pltpu.VMEM((BLK,), jnp.float32), pltpu.VMEM((BLK,), jnp.float32),
        pltpu.VMEM((BLK,), jnp.int32), pltpu.VMEM((BLK,), jnp.int32),
        pltpu.VMEM((ACC_LEN,), jnp.float32),
        pltpu.SemaphoreType.DMA, pltpu.SemaphoreType.DMA,
        pltpu.SemaphoreType.DMA, pltpu.SemaphoreType.DMA,
    ],
)

_pass2 = pl.kernel(
    _pass2_body,
    out_type=jax.ShapeDtypeStruct((1, N_TOTAL), jnp.float32),
    mesh=_MESH,
    compiler_params=_PARAMS,
    scratch_types=[
        pltpu.VMEM((BLK,), jnp.int32), pltpu.VMEM((BLK,), jnp.int32),
        pltpu.VMEM((BLK,), jnp.float32), pltpu.VMEM((BLK,), jnp.float32),
        pltpu.VMEM((PART_LEN,), jnp.float32),
        pltpu.VMEM((L,), jnp.float32),
        pltpu.SemaphoreType.DMA, pltpu.SemaphoreType.DMA,
        pltpu.SemaphoreType.DMA, pltpu.SemaphoreType.DMA,
    ],
)


def kernel(positions, asset_ids, sector_ids, market_caps):
    del asset_ids
    pos_1n = positions.reshape(1, N_TOTAL)
    packed, part = _pass1(pos_1n, sector_ids, market_caps)
    return _pass2(packed, part).reshape(N_TOTAL, 1)


# speedup vs baseline: 32.3321x; 1.5114x over previous
import functools

import jax
import jax.numpy as jnp
from jax import lax
from jax.experimental import pallas as pl
from jax.experimental.pallas import tpu as pltpu
from jax.experimental.pallas import tpu_sc as plsc

N_TOTAL = 5_000_000
NUM_SECTORS = 11
MAX_INDIVIDUAL = 0.01
MAX_SECTOR = 0.15
VOL_SCALAR = 0.15 / 0.2

BLK = 8_192
NBLK = N_TOTAL // BLK
TAIL_BASE = NBLK * BLK
TAIL = N_TOTAL - TAIL_BASE
NC, NS, L = 2, 16, 16
NW = NC * NS
ACC_ROWS = 24
ACC_LEN = ACC_ROWS * L
PART_LEN = NW * ACC_LEN

_MASK_HI = -65536
_MAGIC = 0x5F3759DF


def _rsqrt_newton(x, iters=2):
    bits = lax.bitcast_convert_type(x, jnp.int32)
    y = lax.bitcast_convert_type(_MAGIC - lax.shift_right_logical(bits, 1), jnp.float32)
    th = x * 0.5
    for _ in range(iters):
        y = y * (1.5 - th * y * y)
    return y


NCOPY = 4
GRP = 8


def _num_blocks(wid):
    rem = NBLK % NW
    return jnp.where(wid < rem, NBLK // NW + 1, NBLK // NW)


def _pass1_body(pos_hbm, sec_hbm, mc_hbm, packed_hbm, part_hbm,
                pos0, pos1, sec0, sec1, mc0, mc1, pck0, pck1, acc,
                si0, si1, so0, so1):
    wid = lax.axis_index("s") * NC + lax.axis_index("c")
    zeros = jnp.zeros((L,), jnp.float32)
    for r in range(ACC_ROWS * NCOPY):
        acc[pl.ds(r * L, L)] = zeros
    iota = lax.iota(jnp.int32, L)
    iota_k = [iota + (k % NCOPY) * ACC_LEN for k in range(GRP)]
    slots = ((pos0, sec0, mc0, pck0, si0, so0),
             (pos1, sec1, mc1, pck1, si1, so1))
    nb = _num_blocks(wid)

    def base_of(t):
        return pl.multiple_of((wid + t * NW) * BLK, 128)

    def start_in(t, s):
        b = base_of(t)
        pltpu.async_copy(pos_hbm.at[0, pl.ds(b, BLK)], s[0], s[4])
        pltpu.async_copy(sec_hbm.at[pl.ds(b, BLK)], s[1], s[4])
        pltpu.async_copy(mc_hbm.at[pl.ds(b, BLK)], s[2], s[4])

    def wait_in(s):
        pltpu.make_async_copy(pos_hbm.at[0, pl.ds(0, BLK)], s[0], s[4]).wait()
        pltpu.make_async_copy(sec_hbm.at[pl.ds(0, BLK)], s[1], s[4]).wait()
        pltpu.make_async_copy(mc_hbm.at[pl.ds(0, BLK)], s[2], s[4]).wait()

    def wait_out(s):
        pltpu.make_async_copy(s[3], packed_hbm.at[pl.ds(0, BLK)], s[5]).wait()

    def process(s, nelem):
        pos_v, sec_v, mc_v, pck_v = s[0], s[1], s[2], s[3]

        def chunk(o, k):
            pos = pos_v[pl.ds(o, L)]
            sec = sec_v[pl.ds(o, L)]
            mc = mc_v[pl.ds(o, L)]
            clamped = jnp.clip(pos * VOL_SCALAR, -MAX_INDIVIDUAL, MAX_INDIVIDUAL)
            absc = jnp.abs(clamped)
            x = jnp.maximum(mc, 1e-12)
            sq = x * _rsqrt_newton(x, iters=1)
            u = clamped * sq
            absu = absc * sq
            idx_a = (sec << 4) + iota_k[k]
            plsc.addupdate_scatter(acc, [idx_a], absc)
            plsc.addupdate_scatter(acc, [idx_a + (NUM_SECTORS * L)], absu)
            plsc.addupdate(
                acc.at[pl.ds((k % NCOPY) * ACC_LEN + 2 * NUM_SECTORS * L, L)], mc)
            ub = lax.bitcast_convert_type(u, jnp.int32)
            pck = ((ub + 32768) & _MASK_HI) | sec
            pck_v[pl.ds(o, L)] = pck

        ngroups = nelem // (L * GRP)

        def group(g, carry):
            ob = g * (L * GRP)
            for k in range(GRP):
                chunk(ob + k * L, k)
            return carry

        lax.fori_loop(0, ngroups, group, 0)
        for k in range((nelem - ngroups * L * GRP) // L):
            chunk(ngroups * L * GRP + k * L, k)

    start_in(0, slots[0])

    def blk_body(t, carry):
        for sl in (0, 1):
            @pl.when((t & 1) == sl)
            def _():
                s = slots[sl]
                wait_in(s)

                @pl.when(t + 1 < nb)
                def _():
                    start_in(t + 1, slots[1 - sl])

                @pl.when(t >= 2)
                def _():
                    wait_out(s)

                process(s, BLK)
                pltpu.async_copy(s[3], packed_hbm.at[pl.ds(base_of(t), BLK)], s[5])
        return carry

    lax.fori_loop(0, nb, blk_body, 0)
    wait_out(slots[0])
    wait_out(slots[1])

    @pl.when(wid == NW - 1)
    def _():
        s = slots[0]
        pltpu.sync_copy(pos_hbm.at[0, pl.ds(TAIL_BASE, TAIL)], s[0].at[pl.ds(0, TAIL)])
        pltpu.sync_copy(sec_hbm.at[pl.ds(TAIL_BASE, TAIL)], s[1].at[pl.ds(0, TAIL)])
        pltpu.sync_copy(mc_hbm.at[pl.ds(TAIL_BASE, TAIL)], s[2].at[pl.ds(0, TAIL)])
        process(s, TAIL)
        pltpu.sync_copy(s[3].at[pl.ds(0, TAIL)], packed_hbm.at[pl.ds(TAIL_BASE, TAIL)])

    for r in range(2 * NUM_SECTORS + 1):
        v = acc[pl.ds(r * L, L)]
        for cpy in range(1, NCOPY):
            v = v + acc[pl.ds(cpy * ACC_LEN + r * L, L)]
        acc[pl.ds(r * L, L)] = v
    pltpu.sync_copy(acc.at[pl.ds(0, ACC_LEN)],
                    part_hbm.at[pl.ds(wid * ACC_LEN, ACC_LEN)])


def _pass2_body(packed_hbm, part_hbm, out_hbm,
                pck0, pck1, out0, out1, part_v, lut, si0, si1, so0, so1):
    wid = lax.axis_index("s") * NC + lax.axis_index("c")
    pltpu.sync_copy(part_hbm, part_v)
    iota = lax.iota(jnp.int32, L)
    fiota = iota.astype(jnp.float32)

    rows = []
    for r in range(2 * NUM_SECTORS + 1):
        v = part_v[pl.ds(r * L, L)]
        for w in range(1, NW):
            v = v + part_v[pl.ds(w * ACC_LEN + r * L, L)]
        rows.append(v)

    a_scalars = [jnp.sum(rows[r]) for r in range(NUM_SECTORS)]
    b_scalars = [jnp.sum(rows[NUM_SECTORS + r]) for r in range(NUM_SECTORS)]
    c_total = jnp.sum(rows[2 * NUM_SECTORS])

    zero = jnp.zeros((L,), jnp.float32)
    a_vec = zero
    b_vec = zero
    for r in range(NUM_SECTORS):
        sel = fiota == float(r)
        a_vec = jnp.where(sel, a_scalars[r], a_vec)
        b_vec = jnp.where(sel, b_scalars[r], b_vec)

    scale_v = jnp.where(a_vec > MAX_SECTOR, MAX_SECTOR / a_vec, 1.0)
    c_vec = jnp.maximum(jnp.zeros((L,), jnp.float32) + c_total, 1e-12)
    rc_v = _rsqrt_newton(c_vec)
    total_v = jnp.zeros((L,), jnp.float32) + jnp.sum(scale_v * b_vec * rc_v)
    k_vec = jnp.where(total_v > 1.0, 1.0 / total_v, 1.0)
    lut[...] = scale_v * rc_v * k_vec

    slots = ((pck0, out0, si0, so0), (pck1, out1, si1, so1))
    nb = _num_blocks(wid)

    def base_of(t):
        return pl.multiple_of((wid + t * NW) * BLK, 128)

    def process(s, nelem):
        pck_v, out_v = s[0], s[1]

        def chunk(o):
            x = pck_v[pl.ds(o, L)]
            u = lax.bitcast_convert_type(x & _MASK_HI, jnp.float32)
            sidx = x & 15
            g = plsc.load_gather(lut, [sidx])
            out_v[pl.ds(o, L)] = u * g

        ngroups = nelem // (L * GRP)

        def group(g, carry):
            ob = g * (L * GRP)
            for k in range(GRP):
                chunk(ob + k * L)
            return carry

        lax.fori_loop(0, ngroups, group, 0)
        for k in range((nelem - ngroups * L * GRP) // L):
            chunk(ngroups * L * GRP + k * L)

    def wait_in(s):
        pltpu.make_async_copy(packed_hbm.at[pl.ds(0, BLK)], s[0], s[2]).wait()

    def wait_out(s):
        pltpu.make_async_copy(s[1], out_hbm.at[0, pl.ds(0, BLK)], s[3]).wait()

    pltpu.async_copy(packed_hbm.at[pl.ds(base_of(0), BLK)], slots[0][0], slots[0][2])

    def blk_body(t, carry):
        for sl in (0, 1):
            @pl.when((t & 1) == sl)
            def _():
                s = slots[sl]
                wait_in(s)

                @pl.when(t + 1 < nb)
                def _():
                    sn = slots[1 - sl]
                    pltpu.async_copy(
                        packed_hbm.at[pl.ds(base_of(t + 1), BLK)], sn[0], sn[2])

                @pl.when(t >= 2)
                def _():
                    wait_out(s)

                process(s, BLK)
                pltpu.async_copy(s[1], out_hbm.at[0, pl.ds(base_of(t), BLK)], s[3])
        return carry

    lax.fori_loop(0, nb, blk_body, 0)
    wait_out(slots[0])
    wait_out(slots[1])

    @pl.when(wid == NW - 1)
    def _():
        s = slots[0]
        pltpu.sync_copy(packed_hbm.at[pl.ds(TAIL_BASE, TAIL)], s[0].at[pl.ds(0, TAIL)])
        process(s, TAIL)
        pltpu.sync_copy(s[1].at[pl.ds(0, TAIL)], out_hbm.at[0, pl.ds(TAIL_BASE, TAIL)])


_MESH = plsc.VectorSubcoreMesh(
    core_axis_name="c", subcore_axis_name="s", num_cores=NC, num_subcores=NS)
_PARAMS = pltpu.CompilerParams(needs_layout_passes=False)

_pass1 = pl.kernel(
    _pass1_body,
    out_type=(
        jax.ShapeDtypeStruct((N_TOTAL,), jnp.int32),
        jax.ShapeDtypeStruct((PART_LEN,), jnp.float32),
    ),
    mesh=_MESH,
    compiler_params=_PARAMS,
    scratch_types=[
        pltpu.VMEM((BLK,), jnp.float32), pltpu.VMEM((BLK,), jnp.float32),
        pltpu.VMEM((BLK,), jnp.int32), pltpu.VMEM((BLK,), jnp.int32),
        pltpu.VMEM((BLK,), jnp.float32), pltpu.VMEM((BLK,), jnp.float32),
        pltpu.VMEM((BLK,), jnp.int32), pltpu.VMEM((BLK,), jnp.int32),
        pltpu.VMEM((ACC_LEN * NCOPY,), jnp.float32),
        pltpu.SemaphoreType.DMA, pltpu.SemaphoreType.DMA,
        pltpu.SemaphoreType.DMA, pltpu.SemaphoreType.DMA,
    ],
)

_pass2 = pl.kernel(
    _pass2_body,
    out_type=jax.ShapeDtypeStruct((1, N_TOTAL), jnp.float32),
    mesh=_MESH,
    compiler_params=_PARAMS,
    scratch_types=[
        pltpu.VMEM((BLK,), jnp.int32), pltpu.VMEM((BLK,), jnp.int32),
        pltpu.VMEM((BLK,), jnp.float32), pltpu.VMEM((BLK,), jnp.float32),
        pltpu.VMEM((PART_LEN,), jnp.float32),
        pltpu.VMEM((L,), jnp.float32),
        pltpu.SemaphoreType.DMA, pltpu.SemaphoreType.DMA,
        pltpu.SemaphoreType.DMA, pltpu.SemaphoreType.DMA,
    ],
)


def kernel(positions, asset_ids, sector_ids, market_caps):
    del asset_ids
    pos_1n = positions.reshape(1, N_TOTAL)
    packed, part = _pass1(pos_1n, sector_ids, market_caps)
    return _pass2(packed, part).reshape(N_TOTAL, 1)
